# baseline (speedup 1.0000x reference)
import jax
import jax.numpy as jnp
from jax.experimental import pallas as pl
from jax.experimental.pallas import tpu as pltpu

_BLK = 51200
_SUB = _BLK // 128
_BLKB = 10000


def _assign_body(ann_ref, anc_ref, reg_ref, meta_ref, sums_ref):
    b = pl.program_id(0)
    i = pl.program_id(1)
    M = ann_ref.shape[1]
    A = 100000

    ax1 = anc_ref[0, 0]
    ay1 = anc_ref[1, 0]
    ax2 = anc_ref[2, 0]
    ay2 = anc_ref[3, 0]
    aw = ax2 - ax1
    ah = ay2 - ay1
    acx = ax1 + 0.5 * aw
    acy = ay1 + 0.5 * ah
    area_a = aw * ah

    best = jnp.full(ax1.shape, -1.0, jnp.float32)
    gx1 = jnp.zeros(ax1.shape, jnp.float32)
    gy1 = jnp.zeros(ax1.shape, jnp.float32)
    gx2 = jnp.zeros(ax1.shape, jnp.float32)
    gy2 = jnp.zeros(ax1.shape, jnp.float32)
    id0 = jnp.zeros(ax1.shape, jnp.float32)
    id1 = jnp.zeros(ax1.shape, jnp.float32)
    id2 = jnp.zeros(ax1.shape, jnp.float32)
    for m in range(M):
        bx1 = ann_ref[b, m, 0]
        by1 = ann_ref[b, m, 1]
        bx2 = ann_ref[b, m, 2]
        by2 = ann_ref[b, m, 3]
        area_b = (bx2 - bx1) * (by2 - by1)
        iw = jnp.maximum(jnp.minimum(ax2, bx2) - jnp.maximum(ax1, bx1), 0.0)
        ih = jnp.maximum(jnp.minimum(ay2, by2) - jnp.maximum(ay1, by1), 0.0)
        inter = iw * ih
        ua = jnp.maximum(area_a + area_b - inter, 1e-8)
        iou = inter / ua
        upd = iou > best
        best = jnp.where(upd, iou, best)
        gx1 = jnp.where(upd, bx1, gx1)
        gy1 = jnp.where(upd, by1, gy1)
        gx2 = jnp.where(upd, bx2, gx2)
        gy2 = jnp.where(upd, by2, gy2)
        id0 = jnp.where(upd, ann_ref[b, m, 4], id0)
        id1 = jnp.where(upd, ann_ref[b, m, 5], id1)
        id2 = jnp.where(upd, ann_ref[b, m, 6], id2)

    gidx = (
        i * _BLK
        + jax.lax.broadcasted_iota(jnp.int32, ax1.shape, 0) * 128
        + jax.lax.broadcasted_iota(jnp.int32, ax1.shape, 1)
    )
    valid = gidx < A
    pos = (best >= 0.5) & valid
    wall = (pos | (best < 0.4)) & valid
    posf = pos.astype(jnp.float32)
    num_pos = jnp.sum(posf)

    e = (id0 + 1.0) + 128.0 * (id1 + 1.0) + 16384.0 * (id2 + 1.0)
    meta_ref[0, 0, 0] = wall.astype(jnp.float32)
    meta_ref[0, 1, 0] = jnp.where(pos, e, 0.0)

    gw = gx2 - gx1
    gh = gy2 - gy1
    gcx = gx1 + 0.5 * gw
    gcy = gy1 + 0.5 * gh
    gw = jnp.maximum(gw, 1.0)
    gh = jnp.maximum(gh, 1.0)
    t0 = ((gcx - acx) / aw) / 0.1
    t1 = ((gcy - acy) / ah) / 0.1
    t2 = jnp.log(gw / aw) / 0.2
    t3 = jnp.log(gh / ah) / 0.2

    def sl1(t, k):
        d = jnp.abs(t - reg_ref[0, k, 0])
        return jnp.where(d <= 1.0 / 9.0, 0.5 * 9.0 * d * d, d - 0.5 / 9.0)

    rl = sl1(t0, 0) + sl1(t1, 1) + sl1(t2, 2) + sl1(t3, 3)
    reg_sum = jnp.sum(jnp.where(pos, rl, 0.0))

    base_rows = jnp.concatenate(
        [
            jnp.full((1, 128), num_pos, jnp.float32),
            jnp.full((1, 128), reg_sum, jnp.float32),
            jnp.zeros((6, 128), jnp.float32),
        ],
        axis=0,
    )

    @pl.when(i == 0)
    def _():
        z = -ann_ref[b, 0, 7]
        vz = jnp.full((1, 128), z, jnp.float32)
        sp = jnp.maximum(vz, 0.0) + jnp.log(1.0 + jnp.exp(-jnp.abs(vz)))
        bb = jnp.concatenate(
            [jnp.zeros((2, 128), jnp.float32), sp, jnp.zeros((5, 128), jnp.float32)],
            axis=0,
        )
        sums_ref[0] = base_rows + bb

    @pl.when(i > 0)
    def _():
        sums_ref[0] = sums_ref[0] + base_rows


def _focal_body(fiota_ref, cls_ref, meta_ref, out_ref):
    i = pl.program_id(1)
    C = cls_ref.shape[2]

    fio1 = fiota_ref[0:1, 0:C].astype(jnp.int32) + 1

    mt = meta_ref[0]
    w75 = mt[:, 0:1] * 0.75
    ei = mt[:, 1:2].astype(jnp.int32)
    x = jnp.clip(cls_ref[0], 1e-4, 1.0 - 1e-4)
    oh = (
        ((ei & 127) == fio1)
        | (((ei >> 7) & 127) == fio1)
        | ((ei >> 14) == fio1)
    )
    y = jnp.where(oh, 1.0 - x, x)
    s = jnp.where(oh, 0.25, w75)
    elem = (s * y * y) * (-jnp.log(1.0 - y))
    cls_sum = jnp.sum(elem)

    rows = jnp.concatenate(
        [jnp.full((1, 128), cls_sum, jnp.float32), jnp.zeros((7, 128), jnp.float32)],
        axis=0,
    )

    @pl.when(i == 0)
    def _():
        out_ref[0] = rows

    @pl.when(i > 0)
    def _():
        out_ref[0] = out_ref[0] + rows


def kernel(classifications, regressions, anchors, bbox_exist_prediction, annotations):
    B, A, C = classifications.shape
    M = annotations.shape[1]
    NB = -(-A // _BLK)
    Ap = NB * _BLK

    anc_pack = (
        jnp.pad(anchors[0], ((0, Ap - A), (0, 0)))
        .T.reshape(4, NB, _SUB, 128)
    )
    reg_pack = (
        jnp.pad(regressions, ((0, 0), (0, Ap - A), (0, 0)))
        .transpose(0, 2, 1)
        .reshape(B, 4, NB, _SUB, 128)
    )
    ann_s = jnp.concatenate(
        [annotations, jnp.broadcast_to(bbox_exist_prediction[:, None, :], (B, M, 1))],
        axis=2,
    )

    meta, sums_a = pl.pallas_call(
        _assign_body,
        grid=(B, NB),
        in_specs=[
            pl.BlockSpec(memory_space=pltpu.SMEM),
            pl.BlockSpec((4, 1, _SUB, 128), lambda b, i: (0, i, 0, 0)),
            pl.BlockSpec((1, 4, 1, _SUB, 128), lambda b, i: (b, 0, i, 0, 0)),
        ],
        out_specs=[
            pl.BlockSpec((1, 2, 1, _SUB, 128), lambda b, i: (b, 0, i, 0, 0)),
            pl.BlockSpec((1, 8, 128), lambda b, i: (b, 0, 0)),
        ],
        out_shape=[
            jax.ShapeDtypeStruct((B, 2, NB, _SUB, 128), jnp.float32),
            jax.ShapeDtypeStruct((B, 8, 128), jnp.float32),
        ],
    )(ann_s, anc_pack, reg_pack)

    metaT = meta.reshape(B, 2, Ap).transpose(0, 2, 1)
    fiota = jnp.arange(128, dtype=jnp.float32)[None]

    NBB = A // _BLKB
    out_b = pl.pallas_call(
        _focal_body,
        grid=(B, NBB),
        in_specs=[
            pl.BlockSpec((1, 128), lambda b, i: (0, 0)),
            pl.BlockSpec((1, _BLKB, C), lambda b, i: (b, i, 0)),
            pl.BlockSpec((1, _BLKB, 2), lambda b, i: (b, i, 0)),
        ],
        out_specs=pl.BlockSpec((1, 8, 128), lambda b, i: (b, 0, 0)),
        out_shape=jax.ShapeDtypeStruct((B, 8, 128), jnp.float32),
    )(fiota, classifications, metaT)

    npos = sums_a[:, 0, 0]
    reg_sum = sums_a[:, 1, 0]
    bbox = sums_a[:, 2, 0]
    cls_sum = out_b[:, 0, 0]
    cls_loss = jnp.mean(cls_sum / jnp.maximum(npos, 1.0), keepdims=True)
    reg_loss = jnp.mean(
        jnp.where(npos > 0, reg_sum / jnp.maximum(npos * 4.0, 1.0), 0.0),
        keepdims=True,
    )
    bbox_loss = jnp.mean(bbox, keepdims=True)
    return (cls_loss, reg_loss, bbox_loss)


# speedup vs baseline: 1.0087x; 1.0087x over previous
import jax
import jax.numpy as jnp
from jax.experimental import pallas as pl
from jax.experimental.pallas import tpu as pltpu

_BLK = 25600
_SUB = _BLK // 128
_BLKB = 10000


def _assign_body(ann_ref, anc_ref, reg_ref, meta_ref, sums_ref):
    b = pl.program_id(0)
    i = pl.program_id(1)
    M = ann_ref.shape[1]
    A = 100000

    ax1 = anc_ref[0, 0]
    ay1 = anc_ref[1, 0]
    ax2 = anc_ref[2, 0]
    ay2 = anc_ref[3, 0]
    aw = ax2 - ax1
    ah = ay2 - ay1
    acx = ax1 + 0.5 * aw
    acy = ay1 + 0.5 * ah
    area_a = aw * ah

    best = jnp.full(ax1.shape, -1.0, jnp.float32)
    gx1 = jnp.zeros(ax1.shape, jnp.float32)
    gy1 = jnp.zeros(ax1.shape, jnp.float32)
    gx2 = jnp.zeros(ax1.shape, jnp.float32)
    gy2 = jnp.zeros(ax1.shape, jnp.float32)
    id0 = jnp.zeros(ax1.shape, jnp.float32)
    id1 = jnp.zeros(ax1.shape, jnp.float32)
    id2 = jnp.zeros(ax1.shape, jnp.float32)
    for m in range(M):
        bx1 = ann_ref[b, m, 0]
        by1 = ann_ref[b, m, 1]
        bx2 = ann_ref[b, m, 2]
        by2 = ann_ref[b, m, 3]
        area_b = (bx2 - bx1) * (by2 - by1)
        iw = jnp.maximum(jnp.minimum(ax2, bx2) - jnp.maximum(ax1, bx1), 0.0)
        ih = jnp.maximum(jnp.minimum(ay2, by2) - jnp.maximum(ay1, by1), 0.0)
        inter = iw * ih
        ua = jnp.maximum(area_a + area_b - inter, 1e-8)
        iou = inter / ua
        upd = iou > best
        best = jnp.where(upd, iou, best)
        gx1 = jnp.where(upd, bx1, gx1)
        gy1 = jnp.where(upd, by1, gy1)
        gx2 = jnp.where(upd, bx2, gx2)
        gy2 = jnp.where(upd, by2, gy2)
        id0 = jnp.where(upd, ann_ref[b, m, 4], id0)
        id1 = jnp.where(upd, ann_ref[b, m, 5], id1)
        id2 = jnp.where(upd, ann_ref[b, m, 6], id2)

    gidx = (
        i * _BLK
        + jax.lax.broadcasted_iota(jnp.int32, ax1.shape, 0) * 128
        + jax.lax.broadcasted_iota(jnp.int32, ax1.shape, 1)
    )
    valid = gidx < A
    pos = (best >= 0.5) & valid
    wall = (pos | (best < 0.4)) & valid
    posf = pos.astype(jnp.float32)
    num_pos = jnp.sum(posf)

    e = (id0 + 1.0) + 128.0 * (id1 + 1.0) + 16384.0 * (id2 + 1.0)
    meta_ref[0, 0, 0] = wall.astype(jnp.float32)
    meta_ref[0, 1, 0] = jnp.where(pos, e, 0.0)

    gw = gx2 - gx1
    gh = gy2 - gy1
    gcx = gx1 + 0.5 * gw
    gcy = gy1 + 0.5 * gh
    gw = jnp.maximum(gw, 1.0)
    gh = jnp.maximum(gh, 1.0)
    t0 = ((gcx - acx) / aw) / 0.1
    t1 = ((gcy - acy) / ah) / 0.1
    t2 = jnp.log(gw / aw) / 0.2
    t3 = jnp.log(gh / ah) / 0.2

    def sl1(t, k):
        d = jnp.abs(t - reg_ref[0, k, 0])
        return jnp.where(d <= 1.0 / 9.0, 0.5 * 9.0 * d * d, d - 0.5 / 9.0)

    rl = sl1(t0, 0) + sl1(t1, 1) + sl1(t2, 2) + sl1(t3, 3)
    reg_sum = jnp.sum(jnp.where(pos, rl, 0.0))

    base_rows = jnp.concatenate(
        [
            jnp.full((1, 128), num_pos, jnp.float32),
            jnp.full((1, 128), reg_sum, jnp.float32),
            jnp.zeros((6, 128), jnp.float32),
        ],
        axis=0,
    )

    @pl.when(i == 0)
    def _():
        z = -ann_ref[b, 0, 7]
        vz = jnp.full((1, 128), z, jnp.float32)
        sp = jnp.maximum(vz, 0.0) + jnp.log(1.0 + jnp.exp(-jnp.abs(vz)))
        bb = jnp.concatenate(
            [jnp.zeros((2, 128), jnp.float32), sp, jnp.zeros((5, 128), jnp.float32)],
            axis=0,
        )
        sums_ref[0] = base_rows + bb

    @pl.when(i > 0)
    def _():
        sums_ref[0] = sums_ref[0] + base_rows


def _focal_body(fiota_ref, cls_ref, meta_ref, out_ref):
    i = pl.program_id(1)
    C = cls_ref.shape[2]

    fio1 = fiota_ref[0:1, 0:C].astype(jnp.int32) + 1

    mt = meta_ref[0]
    w75 = mt[:, 0:1] * 0.75
    ei = mt[:, 1:2].astype(jnp.int32)
    x = jnp.clip(cls_ref[0], 1e-4, 1.0 - 1e-4)
    oh = (
        ((ei & 127) == fio1)
        | (((ei >> 7) & 127) == fio1)
        | ((ei >> 14) == fio1)
    )
    y = jnp.where(oh, 1.0 - x, x)
    s = jnp.where(oh, 0.25, w75)
    elem = (s * y * y) * (-jnp.log(1.0 - y))
    cls_sum = jnp.sum(elem)

    rows = jnp.concatenate(
        [jnp.full((1, 128), cls_sum, jnp.float32), jnp.zeros((7, 128), jnp.float32)],
        axis=0,
    )

    @pl.when(i == 0)
    def _():
        out_ref[0] = rows

    @pl.when(i > 0)
    def _():
        out_ref[0] = out_ref[0] + rows


def kernel(classifications, regressions, anchors, bbox_exist_prediction, annotations):
    B, A, C = classifications.shape
    M = annotations.shape[1]
    NB = -(-A // _BLK)
    Ap = NB * _BLK

    anc_pack = (
        jnp.pad(anchors[0], ((0, Ap - A), (0, 0)))
        .T.reshape(4, NB, _SUB, 128)
    )
    reg_pack = (
        jnp.pad(regressions, ((0, 0), (0, Ap - A), (0, 0)))
        .transpose(0, 2, 1)
        .reshape(B, 4, NB, _SUB, 128)
    )
    ann_s = jnp.concatenate(
        [annotations, jnp.broadcast_to(bbox_exist_prediction[:, None, :], (B, M, 1))],
        axis=2,
    )

    meta, sums_a = pl.pallas_call(
        _assign_body,
        grid=(B, NB),
        in_specs=[
            pl.BlockSpec(memory_space=pltpu.SMEM),
            pl.BlockSpec((4, 1, _SUB, 128), lambda b, i: (0, i, 0, 0)),
            pl.BlockSpec((1, 4, 1, _SUB, 128), lambda b, i: (b, 0, i, 0, 0)),
        ],
        out_specs=[
            pl.BlockSpec((1, 2, 1, _SUB, 128), lambda b, i: (b, 0, i, 0, 0)),
            pl.BlockSpec((1, 8, 128), lambda b, i: (b, 0, 0)),
        ],
        out_shape=[
            jax.ShapeDtypeStruct((B, 2, NB, _SUB, 128), jnp.float32),
            jax.ShapeDtypeStruct((B, 8, 128), jnp.float32),
        ],
    )(ann_s, anc_pack, reg_pack)

    metaT = meta.reshape(B, 2, Ap).transpose(0, 2, 1)
    fiota = jnp.arange(128, dtype=jnp.float32)[None]

    NBB = A // _BLKB
    out_b = pl.pallas_call(
        _focal_body,
        grid=(B, NBB),
        in_specs=[
            pl.BlockSpec((1, 128), lambda b, i: (0, 0)),
            pl.BlockSpec((1, _BLKB, C), lambda b, i: (b, i, 0)),
            pl.BlockSpec((1, _BLKB, 2), lambda b, i: (b, i, 0)),
        ],
        out_specs=pl.BlockSpec((1, 8, 128), lambda b, i: (b, 0, 0)),
        out_shape=jax.ShapeDtypeStruct((B, 8, 128), jnp.float32),
    )(fiota, classifications, metaT)

    npos = sums_a[:, 0, 0]
    reg_sum = sums_a[:, 1, 0]
    bbox = sums_a[:, 2, 0]
    cls_sum = out_b[:, 0, 0]
    cls_loss = jnp.mean(cls_sum / jnp.maximum(npos, 1.0), keepdims=True)
    reg_loss = jnp.mean(
        jnp.where(npos > 0, reg_sum / jnp.maximum(npos * 4.0, 1.0), 0.0),
        keepdims=True,
    )
    bbox_loss = jnp.mean(bbox, keepdims=True)
    return (cls_loss, reg_loss, bbox_loss)
